# Initial kernel scaffold; baseline (speedup 1.0000x reference)
import jax
import jax.numpy as jnp
from jax.experimental import pallas as pl


def kernel(x, codebook):
    raise NotImplementedError("write your pallas kernel here")


# speedup vs baseline: 1.2384x; 1.2384x over previous
import functools

import jax
import jax.numpy as jnp
from jax import lax
from jax.experimental import pallas as pl
from jax.experimental.pallas import tpu as pltpu
from jax.experimental.pallas import tpu_sc as plsc

_DIM = 64
_CBS = 8192
_TB = 512
_CBB = 4096
_COMMIT_W = 1.0


def _argmin_body(x_ref, cbt_ref, cbsq_ref, idx_ref, loss_ref,
                 min_s, val_s, arg_s, acc_s):
    t = pl.program_id(0)
    c = pl.program_id(1)
    nt = pl.num_programs(0)
    nc = pl.num_programs(1)

    x = x_ref[...]
    cbt = cbt_ref[...]
    x_sq = jnp.sum(x * x, axis=1, keepdims=True)
    cb_sq = cbsq_ref[...]
    mm = lax.dot_general(x, cbt, (((1,), (0,)), ((), ())),
                         preferred_element_type=jnp.float32)
    dist = (x_sq - 2.0 * mm) + cb_sq

    local_min = jnp.min(dist, axis=1, keepdims=True)
    col = lax.broadcasted_iota(jnp.int32, dist.shape, 1)
    local_arg = jnp.min(jnp.where(dist == local_min, col, _CBS),
                        axis=1, keepdims=True) + c * _CBB

    def _round(v):
        return v.astype(jnp.bfloat16).astype(jnp.float32)

    @pl.when(c == 0)
    def _():
        min_s[...] = _round(local_min)
        val_s[...] = local_min
        arg_s[...] = local_arg

    @pl.when(c > 0)
    def _():
        better = local_min < min_s[...]
        min_s[...] = _round(jnp.where(better, local_min, min_s[...]))
        val_s[...] = jnp.where(better, local_min, val_s[...])
        arg_s[...] = jnp.where(better, local_arg, arg_s[...])

    @pl.when(c == nc - 1)
    def _():
        idx_ref[...] = arg_s[...]
        blk = jnp.sum(val_s[...])
        prev = jnp.where(t == 0, 0.0, acc_s[0])
        tot = prev + blk
        acc_s[0] = tot

        @pl.when(t == nt - 1)
        def _():
            loss_ref[0, 0] = tot * (_COMMIT_W / (nt * _TB * _DIM))


def _argmin_call(flat, codebook):
    ntok = flat.shape[0]
    grid = (ntok // _TB, _CBS // _CBB)
    cbt = codebook.T
    cbsq = jnp.sum(codebook * codebook, axis=1)[None, :]
    return pl.pallas_call(
        _argmin_body,
        grid=grid,
        in_specs=[
            pl.BlockSpec((_TB, _DIM), lambda t, c: (t, 0)),
            pl.BlockSpec((_DIM, _CBB), lambda t, c: (0, c)),
            pl.BlockSpec((1, _CBB), lambda t, c: (0, c)),
        ],
        out_specs=[
            pl.BlockSpec((_TB, 1), lambda t, c: (t, 0)),
            pl.BlockSpec(memory_space=pltpu.SMEM),
        ],
        out_shape=[
            jax.ShapeDtypeStruct((ntok, 1), jnp.int32),
            jax.ShapeDtypeStruct((1, 1), jnp.float32),
        ],
        scratch_shapes=[
            pltpu.VMEM((_TB, 1), jnp.float32),
            pltpu.VMEM((_TB, 1), jnp.float32),
            pltpu.VMEM((_TB, 1), jnp.int32),
            pltpu.SMEM((1,), jnp.float32),
        ],
        compiler_params=pltpu.CompilerParams(
            dimension_semantics=("arbitrary", "arbitrary"),
        ),
    )(flat, cbt, cbsq)


def _make_sc_gather(ntok):
    info = plsc.get_sparse_core_info()
    nw = info.num_cores * info.num_subcores
    bpw = ntok // nw
    mesh = plsc.VectorSubcoreMesh(core_axis_name="c", subcore_axis_name="s")

    @functools.partial(
        pl.kernel,
        mesh=mesh,
        out_type=jax.ShapeDtypeStruct((ntok, _DIM), jnp.float32),
        scratch_types=[
            pltpu.VMEM((bpw,), jnp.int32),
            pltpu.VMEM((bpw, _DIM), jnp.float32),
            pltpu.SemaphoreType.DMA,
        ],
        compiler_params=pltpu.CompilerParams(use_tc_tiling_on_sc=False),
    )
    def gather_k(cb_hbm, idx_hbm, out_hbm, idx_v, rows_v, sem):
        wid = lax.axis_index("s") * info.num_cores + lax.axis_index("c")
        base = wid * bpw
        pltpu.sync_copy(idx_hbm.at[pl.ds(base, bpw)], idx_v)
        pltpu.async_copy(cb_hbm.at[idx_v], rows_v, sem).wait()
        pltpu.sync_copy(rows_v, out_hbm.at[pl.ds(base, bpw)])

    return gather_k


def kernel(x, codebook):
    B, N, D = x.shape
    flat = x.reshape(-1, D)
    idx2d, loss = _argmin_call(flat, codebook)
    indices_flat = idx2d.reshape(-1)
    quantized_flat = _make_sc_gather(flat.shape[0])(codebook, indices_flat)
    quantized = quantized_flat.reshape(B, N, D)
    quantized_st = x + lax.stop_gradient(quantized - x)
    indices = indices_flat.reshape(B, N)
    vq_loss = loss.reshape(())
    return quantized_st, indices, vq_loss


# speedup vs baseline: 1.2965x; 1.0469x over previous
import functools

import jax
import jax.numpy as jnp
from jax import lax
from jax.experimental import pallas as pl
from jax.experimental.pallas import tpu as pltpu
from jax.experimental.pallas import tpu_sc as plsc

_DIM = 64
_CBS = 8192
_TB = 1024
_CBB = 4096
_COMMIT_W = 1.0


def _argmin_body(x_ref, cbt_ref, cbsq_ref, idx_ref, loss_ref,
                 min_s, val_s, arg_s, acc_s):
    t = pl.program_id(0)
    c = pl.program_id(1)
    nt = pl.num_programs(0)
    nc = pl.num_programs(1)

    x = x_ref[...]
    cbt = cbt_ref[...]
    x_sq = jnp.sum(x * x, axis=1, keepdims=True)
    cb_sq = cbsq_ref[...]
    mm = lax.dot_general(x, cbt, (((1,), (0,)), ((), ())),
                         preferred_element_type=jnp.float32)
    dist = (x_sq - 2.0 * mm) + cb_sq

    local_min = jnp.min(dist, axis=1, keepdims=True)
    col = lax.broadcasted_iota(jnp.int32, dist.shape, 1)
    local_arg = jnp.min(jnp.where(dist == local_min, col, _CBS),
                        axis=1, keepdims=True) + c * _CBB

    def _round(v):
        return v.astype(jnp.bfloat16).astype(jnp.float32)

    @pl.when(c == 0)
    def _():
        min_s[...] = _round(local_min)
        val_s[...] = local_min
        arg_s[...] = local_arg

    @pl.when(c > 0)
    def _():
        better = local_min < min_s[...]
        min_s[...] = _round(jnp.where(better, local_min, min_s[...]))
        val_s[...] = jnp.where(better, local_min, val_s[...])
        arg_s[...] = jnp.where(better, local_arg, arg_s[...])

    @pl.when(c == nc - 1)
    def _():
        idx_ref[...] = arg_s[...]
        blk = jnp.sum(val_s[...])
        prev = jnp.where(t == 0, 0.0, acc_s[0])
        tot = prev + blk
        acc_s[0] = tot

        @pl.when(t == nt - 1)
        def _():
            loss_ref[0, 0] = tot * (_COMMIT_W / (nt * _TB * _DIM))


def _argmin_call(flat, codebook):
    ntok = flat.shape[0]
    grid = (ntok // _TB, _CBS // _CBB)
    cbt = codebook.T
    cbsq = jnp.sum(codebook * codebook, axis=1)[None, :]
    return pl.pallas_call(
        _argmin_body,
        grid=grid,
        in_specs=[
            pl.BlockSpec((_TB, _DIM), lambda t, c: (t, 0)),
            pl.BlockSpec((_DIM, _CBB), lambda t, c: (0, c)),
            pl.BlockSpec((1, _CBB), lambda t, c: (0, c)),
        ],
        out_specs=[
            pl.BlockSpec((_TB, 1), lambda t, c: (t, 0)),
            pl.BlockSpec(memory_space=pltpu.SMEM),
        ],
        out_shape=[
            jax.ShapeDtypeStruct((ntok, 1), jnp.int32),
            jax.ShapeDtypeStruct((1, 1), jnp.float32),
        ],
        scratch_shapes=[
            pltpu.VMEM((_TB, 1), jnp.float32),
            pltpu.VMEM((_TB, 1), jnp.float32),
            pltpu.VMEM((_TB, 1), jnp.int32),
            pltpu.SMEM((1,), jnp.float32),
        ],
        compiler_params=pltpu.CompilerParams(
            dimension_semantics=("arbitrary", "arbitrary"),
        ),
    )(flat, cbt, cbsq)


def _make_sc_gather(ntok):
    info = plsc.get_sparse_core_info()
    nw = info.num_cores * info.num_subcores
    bpw = ntok // nw
    mesh = plsc.VectorSubcoreMesh(core_axis_name="c", subcore_axis_name="s")

    @functools.partial(
        pl.kernel,
        mesh=mesh,
        out_type=jax.ShapeDtypeStruct((ntok, _DIM), jnp.float32),
        scratch_types=[
            pltpu.VMEM((bpw,), jnp.int32),
            pltpu.VMEM((bpw, _DIM), jnp.float32),
            pltpu.SemaphoreType.DMA,
        ],
        compiler_params=pltpu.CompilerParams(use_tc_tiling_on_sc=False),
    )
    def gather_k(cb_hbm, idx_hbm, out_hbm, idx_v, rows_v, sem):
        wid = lax.axis_index("s") * info.num_cores + lax.axis_index("c")
        base = wid * bpw
        pltpu.sync_copy(idx_hbm.at[pl.ds(base, bpw)], idx_v)
        pltpu.async_copy(cb_hbm.at[idx_v], rows_v, sem).wait()
        pltpu.sync_copy(rows_v, out_hbm.at[pl.ds(base, bpw)])

    return gather_k


def kernel(x, codebook):
    B, N, D = x.shape
    flat = x.reshape(-1, D)
    idx2d, loss = _argmin_call(flat, codebook)
    indices_flat = idx2d.reshape(-1)
    quantized_flat = _make_sc_gather(flat.shape[0])(codebook, indices_flat)
    quantized = quantized_flat.reshape(B, N, D)
    quantized_st = x + lax.stop_gradient(quantized - x)
    indices = indices_flat.reshape(B, N)
    vq_loss = loss.reshape(())
    return quantized_st, indices, vq_loss


# speedup vs baseline: 1.3137x; 1.0133x over previous
import functools

import jax
import jax.numpy as jnp
from jax import lax
from jax.experimental import pallas as pl
from jax.experimental.pallas import tpu as pltpu
from jax.experimental.pallas import tpu_sc as plsc

_DIM = 64
_CBS = 8192
_TB = 2048
_CBB = 4096
_COMMIT_W = 1.0


def _argmin_body(x_ref, cbt_ref, cbsq_ref, idx_ref, loss_ref,
                 min_s, val_s, arg_s, acc_s):
    t = pl.program_id(0)
    c = pl.program_id(1)
    nt = pl.num_programs(0)
    nc = pl.num_programs(1)

    x = x_ref[...]
    cbt = cbt_ref[...]
    x_sq = jnp.sum(x * x, axis=1, keepdims=True)
    cb_sq = cbsq_ref[...]
    mm = lax.dot_general(x, cbt, (((1,), (0,)), ((), ())),
                         preferred_element_type=jnp.float32)
    dist = (x_sq - 2.0 * mm) + cb_sq

    local_min = jnp.min(dist, axis=1, keepdims=True)
    col = lax.broadcasted_iota(jnp.int32, dist.shape, 1)
    local_arg = jnp.min(jnp.where(dist == local_min, col, _CBS),
                        axis=1, keepdims=True) + c * _CBB

    def _round(v):
        return v.astype(jnp.bfloat16).astype(jnp.float32)

    @pl.when(c == 0)
    def _():
        min_s[...] = _round(local_min)
        val_s[...] = local_min
        arg_s[...] = local_arg

    @pl.when(c > 0)
    def _():
        better = local_min < min_s[...]
        min_s[...] = _round(jnp.where(better, local_min, min_s[...]))
        val_s[...] = jnp.where(better, local_min, val_s[...])
        arg_s[...] = jnp.where(better, local_arg, arg_s[...])

    @pl.when(c == nc - 1)
    def _():
        idx_ref[...] = arg_s[...]
        blk = jnp.sum(val_s[...])
        prev = jnp.where(t == 0, 0.0, acc_s[0])
        tot = prev + blk
        acc_s[0] = tot

        @pl.when(t == nt - 1)
        def _():
            loss_ref[0, 0] = tot * (_COMMIT_W / (nt * _TB * _DIM))


def _argmin_call(flat, codebook):
    ntok = flat.shape[0]
    grid = (ntok // _TB, _CBS // _CBB)
    cbt = codebook.T
    cbsq = jnp.sum(codebook * codebook, axis=1)[None, :]
    return pl.pallas_call(
        _argmin_body,
        grid=grid,
        in_specs=[
            pl.BlockSpec((_TB, _DIM), lambda t, c: (t, 0)),
            pl.BlockSpec((_DIM, _CBB), lambda t, c: (0, c)),
            pl.BlockSpec((1, _CBB), lambda t, c: (0, c)),
        ],
        out_specs=[
            pl.BlockSpec((_TB, 1), lambda t, c: (t, 0)),
            pl.BlockSpec(memory_space=pltpu.SMEM),
        ],
        out_shape=[
            jax.ShapeDtypeStruct((ntok, 1), jnp.int32),
            jax.ShapeDtypeStruct((1, 1), jnp.float32),
        ],
        scratch_shapes=[
            pltpu.VMEM((_TB, 1), jnp.float32),
            pltpu.VMEM((_TB, 1), jnp.float32),
            pltpu.VMEM((_TB, 1), jnp.int32),
            pltpu.SMEM((1,), jnp.float32),
        ],
        compiler_params=pltpu.CompilerParams(
            dimension_semantics=("arbitrary", "arbitrary"),
        ),
    )(flat, cbt, cbsq)


def _make_sc_gather(ntok):
    info = plsc.get_sparse_core_info()
    nw = info.num_cores * info.num_subcores
    bpw = ntok // nw
    mesh = plsc.VectorSubcoreMesh(core_axis_name="c", subcore_axis_name="s")

    @functools.partial(
        pl.kernel,
        mesh=mesh,
        out_type=jax.ShapeDtypeStruct((ntok, _DIM), jnp.float32),
        scratch_types=[
            pltpu.VMEM((bpw,), jnp.int32),
            pltpu.VMEM((bpw, _DIM), jnp.float32),
            pltpu.SemaphoreType.DMA,
        ],
        compiler_params=pltpu.CompilerParams(use_tc_tiling_on_sc=False),
    )
    def gather_k(cb_hbm, idx_hbm, out_hbm, idx_v, rows_v, sem):
        wid = lax.axis_index("s") * info.num_cores + lax.axis_index("c")
        base = wid * bpw
        pltpu.sync_copy(idx_hbm.at[pl.ds(base, bpw)], idx_v)
        pltpu.async_copy(cb_hbm.at[idx_v], rows_v, sem).wait()
        pltpu.sync_copy(rows_v, out_hbm.at[pl.ds(base, bpw)])

    return gather_k


def kernel(x, codebook):
    B, N, D = x.shape
    flat = x.reshape(-1, D)
    idx2d, loss = _argmin_call(flat, codebook)
    indices_flat = idx2d.reshape(-1)
    quantized_flat = _make_sc_gather(flat.shape[0])(codebook, indices_flat)
    quantized = quantized_flat.reshape(B, N, D)
    quantized_st = x + lax.stop_gradient(quantized - x)
    indices = indices_flat.reshape(B, N)
    vq_loss = loss.reshape(())
    return quantized_st, indices, vq_loss
